# baseline (speedup 1.0000x reference)
import jax
import jax.numpy as jnp
from jax import lax
from jax.experimental import pallas as pl
from jax.experimental.pallas import tpu as pltpu
from jax.experimental.pallas import tpu_sc as plsc

BLK_S = 512
_NPAD = 8


def _sc_gather_body(table_hbm, idx_hbm, emb_hbm, idx_v, rows_v, sem):
    s = lax.axis_index("s")

    @pl.when(s == 0)
    def _():
        pltpu.sync_copy(idx_hbm, idx_v)
        pltpu.async_copy(table_hbm.at[idx_v], rows_v, sem).wait()
        pltpu.sync_copy(rows_v, emb_hbm)


def _sc_gather(table, idx_pad):
    V, D = table.shape
    mesh = plsc.VectorSubcoreMesh(
        core_axis_name="c", subcore_axis_name="s", num_cores=1
    )
    run = pl.kernel(
        _sc_gather_body,
        mesh=mesh,
        out_type=jax.ShapeDtypeStruct((_NPAD, D), jnp.float32),
        scratch_types=[
            pltpu.VMEM((_NPAD,), jnp.int32),
            pltpu.VMEM((_NPAD, D), jnp.float32),
            pltpu.SemaphoreType.DMA,
        ],
    )
    return run(table, idx_pad)


def _add_body(blks_per_b, x_ref, emb_ref, o_ref):
    b = pl.program_id(0) // blks_per_b
    o_ref[...] = x_ref[...] + emb_ref[b]


def kernel(x, tissue_vector, registry_tokens):
    B, S, D = x.shape
    idx = tissue_vector[:, 0].astype(jnp.int32)
    idx_pad = jnp.zeros((_NPAD,), jnp.int32).at[:B].set(idx)
    emb = _sc_gather(registry_tokens, idx_pad)
    emb3 = emb.reshape(_NPAD, 1, D)
    x2 = x.reshape(B * S, D)
    blks_per_b = S // BLK_S
    grid = (B * blks_per_b,)
    import functools as _ft
    out = pl.pallas_call(
        _ft.partial(_add_body, blks_per_b),
        grid=grid,
        in_specs=[
            pl.BlockSpec((BLK_S, D), lambda i: (i, 0)),
            pl.BlockSpec((_NPAD, 1, D), lambda i: (0, 0, 0)),
        ],
        out_specs=pl.BlockSpec((BLK_S, D), lambda i: (i, 0)),
        out_shape=jax.ShapeDtypeStruct((B * S, D), x.dtype),
        compiler_params=pltpu.CompilerParams(
            dimension_semantics=("parallel",),
        ),
    )(x2, emb3)
    return out.reshape(B, S, D)


# speedup vs baseline: 1.0172x; 1.0172x over previous
import jax
import jax.numpy as jnp
from jax import lax
from jax.experimental import pallas as pl
from jax.experimental.pallas import tpu as pltpu
from jax.experimental.pallas import tpu_sc as plsc

BLK_S = 1024
_NPAD = 8


def _sc_gather_body(table_hbm, idx_hbm, emb_hbm, idx_v, rows_v, sem):
    s = lax.axis_index("s")

    @pl.when(s == 0)
    def _():
        pltpu.sync_copy(idx_hbm, idx_v)
        pltpu.async_copy(table_hbm.at[idx_v], rows_v, sem).wait()
        pltpu.sync_copy(rows_v, emb_hbm)


def _sc_gather(table, idx_pad):
    V, D = table.shape
    mesh = plsc.VectorSubcoreMesh(
        core_axis_name="c", subcore_axis_name="s", num_cores=1
    )
    run = pl.kernel(
        _sc_gather_body,
        mesh=mesh,
        out_type=jax.ShapeDtypeStruct((_NPAD, D), jnp.float32),
        scratch_types=[
            pltpu.VMEM((_NPAD,), jnp.int32),
            pltpu.VMEM((_NPAD, D), jnp.float32),
            pltpu.SemaphoreType.DMA,
        ],
    )
    return run(table, idx_pad)


def _add_body(blks_per_b, x_ref, emb_ref, o_ref):
    b = pl.program_id(0) // blks_per_b
    o_ref[...] = x_ref[...] + emb_ref[b]


def kernel(x, tissue_vector, registry_tokens):
    B, S, D = x.shape
    idx = tissue_vector[:, 0].astype(jnp.int32)
    idx_pad = jnp.zeros((_NPAD,), jnp.int32).at[:B].set(idx)
    emb = _sc_gather(registry_tokens, idx_pad)
    emb3 = emb.reshape(_NPAD, 1, D)
    x2 = x.reshape(B * S, D)
    blks_per_b = S // BLK_S
    grid = (B * blks_per_b,)
    import functools as _ft
    out = pl.pallas_call(
        _ft.partial(_add_body, blks_per_b),
        grid=grid,
        in_specs=[
            pl.BlockSpec((BLK_S, D), lambda i: (i, 0)),
            pl.BlockSpec((_NPAD, 1, D), lambda i: (0, 0, 0)),
        ],
        out_specs=pl.BlockSpec((BLK_S, D), lambda i: (i, 0)),
        out_shape=jax.ShapeDtypeStruct((B * S, D), x.dtype),
        compiler_params=pltpu.CompilerParams(
            dimension_semantics=("parallel",),
        ),
    )(x2, emb3)
    return out.reshape(B, S, D)


# speedup vs baseline: 1.0227x; 1.0054x over previous
import jax
import jax.numpy as jnp
from jax import lax
from jax.experimental import pallas as pl
from jax.experimental.pallas import tpu as pltpu
from jax.experimental.pallas import tpu_sc as plsc

_NPAD = 8
_CROWS = 1024
_NBUF = 3


def _sc_gather_body(table_hbm, idx_hbm, emb_hbm, idx_v, rows_v, sem):
    s = lax.axis_index("s")

    @pl.when(s == 0)
    def _():
        pltpu.sync_copy(idx_hbm, idx_v)
        pltpu.async_copy(table_hbm.at[idx_v], rows_v, sem).wait()
        pltpu.sync_copy(rows_v, emb_hbm)


def _sc_gather(table, idx_pad):
    V, D = table.shape
    mesh = plsc.VectorSubcoreMesh(
        core_axis_name="c", subcore_axis_name="s", num_cores=1
    )
    run = pl.kernel(
        _sc_gather_body,
        mesh=mesh,
        out_type=jax.ShapeDtypeStruct((_NPAD, D), jnp.float32),
        scratch_types=[
            pltpu.VMEM((_NPAD,), jnp.int32),
            pltpu.VMEM((_NPAD, D), jnp.float32),
            pltpu.SemaphoreType.DMA,
        ],
    )
    return run(table, idx_pad)


def _add_ring_body(nchunks, rows_per_b, x_hbm, emb_ref, o_hbm, buf, lsem, ssem):
    def load(g):
        ph = g % _NBUF
        return pltpu.make_async_copy(
            x_hbm.at[pl.ds(g * _CROWS, _CROWS), :],
            buf.at[ph],
            lsem.at[ph],
        )

    def store(g):
        ph = g % _NBUF
        return pltpu.make_async_copy(
            buf.at[ph],
            o_hbm.at[pl.ds(g * _CROWS, _CROWS), :],
            ssem.at[ph],
        )

    load(0).start()
    load(1).start()
    for g in range(nchunks):
        ph = g % _NBUF
        load(g).wait()
        b = (g * _CROWS) // rows_per_b
        buf[ph] = buf[ph] + emb_ref[b]
        store(g).start()
        if g + 2 < nchunks:
            if g >= 1:
                store(g - 1).wait()
            load(g + 2).start()
    for g in range(max(nchunks - 3, 0), nchunks):
        store(g).wait()


def kernel(x, tissue_vector, registry_tokens):
    B, S, D = x.shape
    idx = tissue_vector[:, 0].astype(jnp.int32)
    idx_pad = jnp.zeros((_NPAD,), jnp.int32).at[:B].set(idx)
    emb = _sc_gather(registry_tokens, idx_pad)
    x2 = x.reshape(B * S, D)
    nchunks = (B * S) // _CROWS
    rows_per_b = S
    import functools as _ft
    out = pl.pallas_call(
        _ft.partial(_add_ring_body, nchunks, rows_per_b),
        in_specs=[
            pl.BlockSpec(memory_space=pl.ANY),
            pl.BlockSpec((_NPAD, D), lambda: (0, 0)),
        ],
        out_specs=pl.BlockSpec(memory_space=pl.ANY),
        out_shape=jax.ShapeDtypeStruct((B * S, D), x.dtype),
        scratch_shapes=[
            pltpu.VMEM((_NBUF, _CROWS, D), jnp.float32),
            pltpu.SemaphoreType.DMA((_NBUF,)),
            pltpu.SemaphoreType.DMA((_NBUF,)),
        ],
    )(x2, emb)
    return out.reshape(B, S, D)


# speedup vs baseline: 1.0252x; 1.0025x over previous
import jax
import jax.numpy as jnp
from jax import lax
from jax.experimental import pallas as pl
from jax.experimental.pallas import tpu as pltpu
from jax.experimental.pallas import tpu_sc as plsc

_NPAD = 8
_CROWS = 1024
_NBUF = 4


def _sc_gather_body(table_hbm, idx_hbm, emb_hbm, idx_v, rows_v, sem):
    s = lax.axis_index("s")

    @pl.when(s == 0)
    def _():
        pltpu.sync_copy(idx_hbm, idx_v)
        pltpu.async_copy(table_hbm.at[idx_v], rows_v, sem).wait()
        pltpu.sync_copy(rows_v, emb_hbm)


def _sc_gather(table, idx_pad):
    V, D = table.shape
    mesh = plsc.VectorSubcoreMesh(
        core_axis_name="c", subcore_axis_name="s", num_cores=1
    )
    run = pl.kernel(
        _sc_gather_body,
        mesh=mesh,
        out_type=jax.ShapeDtypeStruct((_NPAD, D), jnp.float32),
        scratch_types=[
            pltpu.VMEM((_NPAD,), jnp.int32),
            pltpu.VMEM((_NPAD, D), jnp.float32),
            pltpu.SemaphoreType.DMA,
        ],
    )
    return run(table, idx_pad)


def _add_ring_body(nchunks, rows_per_b, x_hbm, emb_ref, o_hbm, buf, lsem, ssem):
    def load(g):
        ph = g % _NBUF
        return pltpu.make_async_copy(
            x_hbm.at[pl.ds(g * _CROWS, _CROWS), :],
            buf.at[ph],
            lsem.at[ph],
        )

    def store(g):
        ph = g % _NBUF
        return pltpu.make_async_copy(
            buf.at[ph],
            o_hbm.at[pl.ds(g * _CROWS, _CROWS), :],
            ssem.at[ph],
        )

    look = _NBUF - 1
    for g in range(min(look, nchunks)):
        load(g).start()
    for g in range(nchunks):
        ph = g % _NBUF
        load(g).wait()
        b = (g * _CROWS) // rows_per_b
        buf[ph] = buf[ph] + emb_ref[b]
        store(g).start()
        if g + look < nchunks:
            if g >= 1:
                store(g - 1).wait()
            load(g + look).start()
    for g in range(max(nchunks - _NBUF, 0), nchunks):
        store(g).wait()


def kernel(x, tissue_vector, registry_tokens):
    B, S, D = x.shape
    idx = tissue_vector[:, 0].astype(jnp.int32)
    idx_pad = jnp.zeros((_NPAD,), jnp.int32).at[:B].set(idx)
    emb = _sc_gather(registry_tokens, idx_pad)
    x2 = x.reshape(B * S, D)
    nchunks = (B * S) // _CROWS
    rows_per_b = S
    import functools as _ft
    out = pl.pallas_call(
        _ft.partial(_add_ring_body, nchunks, rows_per_b),
        in_specs=[
            pl.BlockSpec(memory_space=pl.ANY),
            pl.BlockSpec((_NPAD, D), lambda: (0, 0)),
        ],
        out_specs=pl.BlockSpec(memory_space=pl.ANY),
        out_shape=jax.ShapeDtypeStruct((B * S, D), x.dtype),
        scratch_shapes=[
            pltpu.VMEM((_NBUF, _CROWS, D), jnp.float32),
            pltpu.SemaphoreType.DMA((_NBUF,)),
            pltpu.SemaphoreType.DMA((_NBUF,)),
        ],
    )(x2, emb)
    return out.reshape(B, S, D)


# speedup vs baseline: 1.0255x; 1.0003x over previous
import jax
import jax.numpy as jnp
from jax import lax
from jax.experimental import pallas as pl
from jax.experimental.pallas import tpu as pltpu
from jax.experimental.pallas import tpu_sc as plsc

_NPAD = 8
_CROWS = 512
_NBUF = 6


def _sc_gather_body(table_hbm, idx_hbm, emb_hbm, idx_v, rows_v, sem):
    s = lax.axis_index("s")

    @pl.when(s == 0)
    def _():
        pltpu.sync_copy(idx_hbm, idx_v)
        pltpu.async_copy(table_hbm.at[idx_v], rows_v, sem).wait()
        pltpu.sync_copy(rows_v, emb_hbm)


def _sc_gather(table, idx_pad):
    V, D = table.shape
    mesh = plsc.VectorSubcoreMesh(
        core_axis_name="c", subcore_axis_name="s", num_cores=1
    )
    run = pl.kernel(
        _sc_gather_body,
        mesh=mesh,
        out_type=jax.ShapeDtypeStruct((_NPAD, D), jnp.float32),
        scratch_types=[
            pltpu.VMEM((_NPAD,), jnp.int32),
            pltpu.VMEM((_NPAD, D), jnp.float32),
            pltpu.SemaphoreType.DMA,
        ],
    )
    return run(table, idx_pad)


def _add_ring_body(nchunks, rows_per_b, x_hbm, emb_ref, o_hbm, buf, lsem, ssem):
    def load(g):
        ph = g % _NBUF
        return pltpu.make_async_copy(
            x_hbm.at[pl.ds(g * _CROWS, _CROWS), :],
            buf.at[ph],
            lsem.at[ph],
        )

    def store(g):
        ph = g % _NBUF
        return pltpu.make_async_copy(
            buf.at[ph],
            o_hbm.at[pl.ds(g * _CROWS, _CROWS), :],
            ssem.at[ph],
        )

    look = _NBUF - 1
    for g in range(min(look, nchunks)):
        load(g).start()
    for g in range(nchunks):
        ph = g % _NBUF
        load(g).wait()
        b = (g * _CROWS) // rows_per_b
        buf[ph] = buf[ph] + emb_ref[b]
        store(g).start()
        if g + look < nchunks:
            if g >= 1:
                store(g - 1).wait()
            load(g + look).start()
    for g in range(max(nchunks - _NBUF, 0), nchunks):
        store(g).wait()


def kernel(x, tissue_vector, registry_tokens):
    B, S, D = x.shape
    idx = tissue_vector[:, 0].astype(jnp.int32)
    idx_pad = jnp.zeros((_NPAD,), jnp.int32).at[:B].set(idx)
    emb = _sc_gather(registry_tokens, idx_pad)
    x2 = x.reshape(B * S, D)
    nchunks = (B * S) // _CROWS
    rows_per_b = S
    import functools as _ft
    out = pl.pallas_call(
        _ft.partial(_add_ring_body, nchunks, rows_per_b),
        in_specs=[
            pl.BlockSpec(memory_space=pl.ANY),
            pl.BlockSpec((_NPAD, D), lambda: (0, 0)),
        ],
        out_specs=pl.BlockSpec(memory_space=pl.ANY),
        out_shape=jax.ShapeDtypeStruct((B * S, D), x.dtype),
        scratch_shapes=[
            pltpu.VMEM((_NBUF, _CROWS, D), jnp.float32),
            pltpu.SemaphoreType.DMA((_NBUF,)),
            pltpu.SemaphoreType.DMA((_NBUF,)),
        ],
    )(x2, emb)
    return out.reshape(B, S, D)


# speedup vs baseline: 1.0271x; 1.0015x over previous
import jax
import jax.numpy as jnp
from jax import lax
from jax.experimental import pallas as pl
from jax.experimental.pallas import tpu as pltpu
from jax.experimental.pallas import tpu_sc as plsc

_NPAD = 8
_CROWS = 2048
_NBUF = 3


def _sc_gather_body(table_hbm, idx_hbm, emb_hbm, idx_v, rows_v, sem):
    s = lax.axis_index("s")

    @pl.when(s == 0)
    def _():
        pltpu.sync_copy(idx_hbm, idx_v)
        pltpu.async_copy(table_hbm.at[idx_v], rows_v, sem).wait()
        pltpu.sync_copy(rows_v, emb_hbm)


def _sc_gather(table, idx_pad):
    V, D = table.shape
    mesh = plsc.VectorSubcoreMesh(
        core_axis_name="c", subcore_axis_name="s", num_cores=1
    )
    run = pl.kernel(
        _sc_gather_body,
        mesh=mesh,
        out_type=jax.ShapeDtypeStruct((_NPAD, D), jnp.float32),
        scratch_types=[
            pltpu.VMEM((_NPAD,), jnp.int32),
            pltpu.VMEM((_NPAD, D), jnp.float32),
            pltpu.SemaphoreType.DMA,
        ],
    )
    return run(table, idx_pad)


def _add_ring_body(nchunks, rows_per_b, x_hbm, emb_ref, o_hbm, buf, lsem, ssem):
    def load(g):
        ph = g % _NBUF
        return pltpu.make_async_copy(
            x_hbm.at[pl.ds(g * _CROWS, _CROWS), :],
            buf.at[ph],
            lsem.at[ph],
        )

    def store(g):
        ph = g % _NBUF
        return pltpu.make_async_copy(
            buf.at[ph],
            o_hbm.at[pl.ds(g * _CROWS, _CROWS), :],
            ssem.at[ph],
        )

    look = _NBUF - 1
    for g in range(min(look, nchunks)):
        load(g).start()
    for g in range(nchunks):
        ph = g % _NBUF
        load(g).wait()
        b = (g * _CROWS) // rows_per_b
        buf[ph] = buf[ph] + emb_ref[b]
        store(g).start()
        if g + look < nchunks:
            if g >= 1:
                store(g - 1).wait()
            load(g + look).start()
    for g in range(max(nchunks - _NBUF, 0), nchunks):
        store(g).wait()


def kernel(x, tissue_vector, registry_tokens):
    B, S, D = x.shape
    idx = tissue_vector[:, 0].astype(jnp.int32)
    idx_pad = jnp.zeros((_NPAD,), jnp.int32).at[:B].set(idx)
    emb = _sc_gather(registry_tokens, idx_pad)
    x2 = x.reshape(B * S, D)
    nchunks = (B * S) // _CROWS
    rows_per_b = S
    import functools as _ft
    out = pl.pallas_call(
        _ft.partial(_add_ring_body, nchunks, rows_per_b),
        in_specs=[
            pl.BlockSpec(memory_space=pl.ANY),
            pl.BlockSpec((_NPAD, D), lambda: (0, 0)),
        ],
        out_specs=pl.BlockSpec(memory_space=pl.ANY),
        out_shape=jax.ShapeDtypeStruct((B * S, D), x.dtype),
        scratch_shapes=[
            pltpu.VMEM((_NBUF, _CROWS, D), jnp.float32),
            pltpu.SemaphoreType.DMA((_NBUF,)),
            pltpu.SemaphoreType.DMA((_NBUF,)),
        ],
    )(x2, emb)
    return out.reshape(B, S, D)


# speedup vs baseline: 1.0341x; 1.0068x over previous
import jax
import jax.numpy as jnp
from jax import lax
from jax.experimental import pallas as pl
from jax.experimental.pallas import tpu as pltpu
from jax.experimental.pallas import tpu_sc as plsc

_NPAD = 4
_CROWS = 2048
_NBUF = 3


def _sc_gather_body(table_hbm, idx_hbm, emb_hbm, idx_v, rows_v, sem):
    s = lax.axis_index("s")

    @pl.when(s == 0)
    def _():
        pltpu.sync_copy(idx_hbm, idx_v)
        pltpu.async_copy(table_hbm.at[idx_v], rows_v, sem).wait()
        pltpu.sync_copy(rows_v, emb_hbm)


def _sc_gather(table, idx_pad):
    V, D = table.shape
    mesh = plsc.VectorSubcoreMesh(
        core_axis_name="c", subcore_axis_name="s", num_cores=1
    )
    run = pl.kernel(
        _sc_gather_body,
        mesh=mesh,
        out_type=jax.ShapeDtypeStruct((_NPAD, D), jnp.float32),
        scratch_types=[
            pltpu.VMEM((_NPAD,), jnp.int32),
            pltpu.VMEM((_NPAD, D), jnp.float32),
            pltpu.SemaphoreType.DMA,
        ],
    )
    return run(table, idx_pad)


def _add_ring_body(nchunks, rows_per_b, x_hbm, emb_ref, o_hbm, buf, lsem, ssem):
    def load(g):
        ph = g % _NBUF
        return pltpu.make_async_copy(
            x_hbm.at[pl.ds(g * _CROWS, _CROWS), :],
            buf.at[ph],
            lsem.at[ph],
        )

    def store(g):
        ph = g % _NBUF
        return pltpu.make_async_copy(
            buf.at[ph],
            o_hbm.at[pl.ds(g * _CROWS, _CROWS), :],
            ssem.at[ph],
        )

    look = _NBUF - 1
    for g in range(min(look, nchunks)):
        load(g).start()
    for g in range(nchunks):
        ph = g % _NBUF
        load(g).wait()
        b = (g * _CROWS) // rows_per_b
        buf[ph] = buf[ph] + emb_ref[b]
        store(g).start()
        if g + look < nchunks:
            if g >= 1:
                store(g - 1).wait()
            load(g + look).start()
    for g in range(max(nchunks - _NBUF, 0), nchunks):
        store(g).wait()


def kernel(x, tissue_vector, registry_tokens):
    B, S, D = x.shape
    idx = tissue_vector[:, 0].astype(jnp.int32)
    idx_pad = jnp.zeros((_NPAD,), jnp.int32).at[:B].set(idx)
    emb = _sc_gather(registry_tokens, idx_pad)
    x2 = x.reshape(B * S, D)
    nchunks = (B * S) // _CROWS
    rows_per_b = S
    import functools as _ft
    out = pl.pallas_call(
        _ft.partial(_add_ring_body, nchunks, rows_per_b),
        in_specs=[
            pl.BlockSpec(memory_space=pl.ANY),
            pl.BlockSpec((_NPAD, D), lambda: (0, 0)),
        ],
        out_specs=pl.BlockSpec(memory_space=pl.ANY),
        out_shape=jax.ShapeDtypeStruct((B * S, D), x.dtype),
        scratch_shapes=[
            pltpu.VMEM((_NBUF, _CROWS, D), jnp.float32),
            pltpu.SemaphoreType.DMA((_NBUF,)),
            pltpu.SemaphoreType.DMA((_NBUF,)),
        ],
    )(x2, emb)
    return out.reshape(B, S, D)
